# baseline (speedup 1.0000x reference)
import functools

import jax
import jax.numpy as jnp
from jax import lax
from jax.experimental import pallas as pl
from jax.experimental.pallas import tpu as pltpu
from jax.experimental.pallas import tpu_sc as plsc

B = 512
N_ATOMS = 2048
N_ANDS = 4096
AND_SIZE = 8
N_ORS = 512
OR_SIZE = 8

NC = 2
NS = 16
NW = NC * NS
L = 16

NCHUNK = 8
ANDS_PER_CHUNK = N_ANDS // NCHUNK
GPC = ANDS_PER_CHUNK // L
GOR = N_ORS // L

_mesh = plsc.VectorSubcoreMesh(core_axis_name="c", subcore_axis_name="s")


def _tree_prod(vals):
    while len(vals) > 1:
        nxt = [vals[i] * vals[i + 1] for i in range(0, len(vals) - 1, 2)]
        if len(vals) % 2:
            nxt.append(vals[-1])
        vals = nxt
    return vals[0]


@functools.partial(
    pl.kernel,
    out_type=jax.ShapeDtypeStruct((B, N_ORS), jnp.float32),
    mesh=_mesh,
    compiler_params=pltpu.CompilerParams(needs_layout_passes=False),
    scratch_types=[
        pltpu.VMEM((L * N_ATOMS,), jnp.float32),
        pltpu.VMEM((L * N_ANDS,), jnp.float32),
        pltpu.VMEM((2, AND_SIZE, ANDS_PER_CHUNK), jnp.int32),
        pltpu.VMEM((OR_SIZE, N_ORS), jnp.int32),
        pltpu.VMEM((L, N_ORS), jnp.float32),
        pltpu.SemaphoreType.DMA,
        pltpu.SemaphoreType.DMA,
        pltpu.SemaphoreType.DMA,
    ],
)
def _reasoner(x_hbm, ands_hbm, ors_hbm, out_hbm,
              x_v, ae_v, andsc_v, ors_v, out_v, sem_x, sem_a, sem_b):
    wid = lax.axis_index("s") * NC + lax.axis_index("c")

    cp_x = pltpu.async_copy(
        x_hbm.at[pl.ds(wid * (L * N_ATOMS), L * N_ATOMS)], x_v, sem_x)
    cp_c = [pltpu.async_copy(ands_hbm.at[0], andsc_v.at[0], sem_a),
            pltpu.async_copy(ands_hbm.at[1], andsc_v.at[1], sem_b)]
    pltpu.sync_copy(ors_hbm, ors_v)
    cp_x.wait()

    for c in range(NCHUNK):
        buf = c % 2
        cp_c[buf].wait()

        def make_group(c, buf):
            def group(g):
                goff = pl.multiple_of(g * L, L)
                idx = [andsc_v[buf, j, pl.ds(goff, L)]
                       for j in range(AND_SIZE)]
                for l in range(L):
                    xs = x_v.at[pl.ds(l * N_ATOMS, N_ATOMS)]
                    acc = _tree_prod(
                        [plsc.load_gather(xs, [idx[j]])
                         for j in range(AND_SIZE)])
                    ae_v[pl.ds(l * N_ANDS + c * ANDS_PER_CHUNK + goff, L)] = acc
            return group

        def wrap(g, carry, c=c, buf=buf):
            make_group(c, buf)(g)
            return carry
        lax.fori_loop(0, GPC, wrap, 0)
        if c + 2 < NCHUNK:
            cp_c[buf] = pltpu.async_copy(
                ands_hbm.at[c + 2], andsc_v.at[buf],
                sem_a if buf == 0 else sem_b)

    def group2(g, carry):
        goff = pl.multiple_of(g * L, L)
        idx = [ors_v[j, pl.ds(goff, L)] for j in range(OR_SIZE)]
        for l in range(L):
            aes = ae_v.at[pl.ds(l * N_ANDS, N_ANDS)]
            acc = _tree_prod(
                [1.0 - plsc.load_gather(aes, [idx[j]])
                 for j in range(OR_SIZE)])
            out_v[l, pl.ds(goff, L)] = 1.0 - acc
        return carry

    lax.fori_loop(0, GOR, group2, 0)
    pltpu.sync_copy(out_v, out_hbm.at[pl.ds(wid * L, L)])


def kernel(x, exp, log, ands, ors):
    del exp, log
    xf = jnp.reshape(x, (-1,))
    ands_c = jnp.transpose(
        jnp.reshape(jnp.transpose(ands), (AND_SIZE, NCHUNK, ANDS_PER_CHUNK)),
        (1, 0, 2))
    orsT = jnp.transpose(ors)
    return _reasoner(xf, ands_c, orsT)


# speedup vs baseline: 1.4630x; 1.4630x over previous
import functools

import jax
import jax.numpy as jnp
from jax import lax
from jax.experimental import pallas as pl
from jax.experimental.pallas import tpu as pltpu
from jax.experimental.pallas import tpu_sc as plsc

B = 512
N_ATOMS = 2048
N_ANDS = 4096
AND_SIZE = 8
N_ORS = 512
OR_SIZE = 8

NC = 2
NS = 16
NW = NC * NS
L = 16

NCHUNK = 8
ANDS_PER_CHUNK = N_ANDS // NCHUNK
GPC = ANDS_PER_CHUNK // L
GOR = N_ORS // L

_mesh = plsc.VectorSubcoreMesh(core_axis_name="c", subcore_axis_name="s")


def _tree_prod(vals):
    while len(vals) > 1:
        nxt = [vals[i] * vals[i + 1] for i in range(0, len(vals) - 1, 2)]
        if len(vals) % 2:
            nxt.append(vals[-1])
        vals = nxt
    return vals[0]


@functools.partial(
    pl.kernel,
    out_type=jax.ShapeDtypeStruct((B, N_ORS), jnp.float32),
    mesh=_mesh,
    compiler_params=pltpu.CompilerParams(needs_layout_passes=False),
    scratch_types=[
        pltpu.VMEM((L * N_ATOMS,), jnp.float32),
        pltpu.VMEM((L * N_ANDS,), jnp.float32),
        pltpu.VMEM((2, AND_SIZE, ANDS_PER_CHUNK), jnp.int32),
        pltpu.VMEM((OR_SIZE, N_ORS), jnp.int32),
        pltpu.VMEM((L, N_ORS), jnp.float32),
        pltpu.SemaphoreType.DMA,
        pltpu.SemaphoreType.DMA,
        pltpu.SemaphoreType.DMA,
    ],
)
def _reasoner(x_hbm, ands_hbm, ors_hbm, out_hbm,
              x_v, ae_v, andsc_v, ors_v, out_v, sem_x, sem_a, sem_b):
    wid = lax.axis_index("s") * NC + lax.axis_index("c")

    cp_x = pltpu.async_copy(
        x_hbm.at[pl.ds(wid * (L * N_ATOMS), L * N_ATOMS)], x_v, sem_x)
    cp_c = [pltpu.async_copy(ands_hbm.at[0], andsc_v.at[0], sem_a),
            pltpu.async_copy(ands_hbm.at[1], andsc_v.at[1], sem_b)]
    pltpu.sync_copy(ors_hbm, ors_v)
    cp_x.wait()

    for c in range(NCHUNK):
        buf = c % 2
        cp_c[buf].wait()

        def make_group(c, buf):
            def group(g):
                goff = pl.multiple_of(g * L, L)
                idx = [andsc_v[buf, j, pl.ds(goff, L)]
                       for j in range(AND_SIZE)]
                accs = []
                for l in range(L):
                    xs = x_v.at[pl.ds(l * N_ATOMS, N_ATOMS)]
                    accs.append(_tree_prod(
                        [plsc.load_gather(xs, [idx[j]])
                         for j in range(AND_SIZE)]))
                for l in range(L):
                    ae_v[pl.ds(l * N_ANDS + c * ANDS_PER_CHUNK + goff, L)] = accs[l]
            return group

        def wrap(g, carry, c=c, buf=buf):
            make_group(c, buf)(g)
            return carry
        lax.fori_loop(0, GPC, wrap, 0)
        if c + 2 < NCHUNK:
            cp_c[buf] = pltpu.async_copy(
                ands_hbm.at[c + 2], andsc_v.at[buf],
                sem_a if buf == 0 else sem_b)

    def group2(g, carry):
        goff = pl.multiple_of(g * L, L)
        idx = [ors_v[j, pl.ds(goff, L)] for j in range(OR_SIZE)]
        accs = []
        for l in range(L):
            aes = ae_v.at[pl.ds(l * N_ANDS, N_ANDS)]
            accs.append(_tree_prod(
                [1.0 - plsc.load_gather(aes, [idx[j]])
                 for j in range(OR_SIZE)]))
        for l in range(L):
            out_v[l, pl.ds(goff, L)] = 1.0 - accs[l]
        return carry

    lax.fori_loop(0, GOR, group2, 0)
    pltpu.sync_copy(out_v, out_hbm.at[pl.ds(wid * L, L)])


def kernel(x, exp, log, ands, ors):
    del exp, log
    xf = jnp.reshape(x, (-1,))
    ands_c = jnp.transpose(
        jnp.reshape(jnp.transpose(ands), (AND_SIZE, NCHUNK, ANDS_PER_CHUNK)),
        (1, 0, 2))
    orsT = jnp.transpose(ors)
    return _reasoner(xf, ands_c, orsT)


# speedup vs baseline: 1.4891x; 1.0178x over previous
import functools

import jax
import jax.numpy as jnp
from jax import lax
from jax.experimental import pallas as pl
from jax.experimental.pallas import tpu as pltpu
from jax.experimental.pallas import tpu_sc as plsc

B = 512
N_ATOMS = 2048
N_ANDS = 4096
AND_SIZE = 8
N_ORS = 512
OR_SIZE = 8

NC = 2
NS = 16
NW = NC * NS
L = 16

NCHUNK = 8
ANDS_PER_CHUNK = N_ANDS // NCHUNK
GPC = ANDS_PER_CHUNK // L
GOR = N_ORS // L

_mesh = plsc.VectorSubcoreMesh(core_axis_name="c", subcore_axis_name="s")


def _tree_prod(vals):
    while len(vals) > 1:
        nxt = [vals[i] * vals[i + 1] for i in range(0, len(vals) - 1, 2)]
        if len(vals) % 2:
            nxt.append(vals[-1])
        vals = nxt
    return vals[0]


@functools.partial(
    pl.kernel,
    out_type=jax.ShapeDtypeStruct((B, N_ORS), jnp.float32),
    mesh=_mesh,
    compiler_params=pltpu.CompilerParams(needs_layout_passes=False),
    scratch_types=[
        pltpu.VMEM((L * N_ATOMS,), jnp.float32),
        pltpu.VMEM((L * N_ANDS,), jnp.float32),
        pltpu.VMEM((2, AND_SIZE, ANDS_PER_CHUNK), jnp.int32),
        pltpu.VMEM((OR_SIZE, N_ORS), jnp.int32),
        pltpu.VMEM((L, N_ORS), jnp.float32),
        pltpu.SemaphoreType.DMA,
        pltpu.SemaphoreType.DMA,
        pltpu.SemaphoreType.DMA,
    ],
)
def _reasoner(x_hbm, ands_hbm, ors_hbm, out_hbm,
              x_v, ae_v, andsc_v, ors_v, out_v, sem_x, sem_a, sem_b):
    wid = lax.axis_index("s") * NC + lax.axis_index("c")

    cp_x = pltpu.async_copy(
        x_hbm.at[pl.ds(wid * (L * N_ATOMS), L * N_ATOMS)], x_v, sem_x)
    cp_c = [pltpu.async_copy(ands_hbm.at[0], andsc_v.at[0], sem_a),
            pltpu.async_copy(ands_hbm.at[1], andsc_v.at[1], sem_b)]
    pltpu.sync_copy(ors_hbm, ors_v)
    cp_x.wait()

    for c in range(NCHUNK):
        buf = c % 2
        cp_c[buf].wait()

        def make_group(c, buf):
            def group(g):
                goff = pl.multiple_of(g * L, L)
                idx = [andsc_v[buf, j, pl.ds(goff, L)]
                       for j in range(AND_SIZE)]
                accs = []
                for l in range(L):
                    xs = x_v.at[pl.ds(l * N_ATOMS, N_ATOMS)]
                    accs.append(_tree_prod(
                        [plsc.load_gather(xs, [idx[j]])
                         for j in range(AND_SIZE)]))
                for l in range(L):
                    ae_v[pl.ds(l * N_ANDS + c * ANDS_PER_CHUNK + goff, L)] = accs[l]
            return group

        plsc.parallel_loop(0, GPC, 1, unroll=1)(make_group(c, buf))
        if c + 2 < NCHUNK:
            cp_c[buf] = pltpu.async_copy(
                ands_hbm.at[c + 2], andsc_v.at[buf],
                sem_a if buf == 0 else sem_b)

    def group2(g):
        goff = pl.multiple_of(g * L, L)
        idx = [ors_v[j, pl.ds(goff, L)] for j in range(OR_SIZE)]
        accs = []
        for l in range(L):
            aes = ae_v.at[pl.ds(l * N_ANDS, N_ANDS)]
            accs.append(_tree_prod(
                [1.0 - plsc.load_gather(aes, [idx[j]])
                 for j in range(OR_SIZE)]))
        for l in range(L):
            out_v[l, pl.ds(goff, L)] = 1.0 - accs[l]

    plsc.parallel_loop(0, GOR, 1, unroll=1)(group2)
    pltpu.sync_copy(out_v, out_hbm.at[pl.ds(wid * L, L)])


def kernel(x, exp, log, ands, ors):
    del exp, log
    xf = jnp.reshape(x, (-1,))
    ands_c = jnp.transpose(
        jnp.reshape(jnp.transpose(ands), (AND_SIZE, NCHUNK, ANDS_PER_CHUNK)),
        (1, 0, 2))
    orsT = jnp.transpose(ors)
    return _reasoner(xf, ands_c, orsT)
